# baseline (speedup 1.0000x reference)
import functools

import jax
import jax.numpy as jnp
from jax import lax
from jax.experimental import pallas as pl
from jax.experimental.pallas import tpu as pltpu
from jax.experimental.pallas import tpu_sc as plsc

BATCH = 4096
HIST = 50
EMBED = 128
NUM_WORKERS = 32
COLS_PER_W = BATCH // NUM_WORKERS
NBUF = 3

_mesh = plsc.VectorSubcoreMesh(core_axis_name="c", subcore_axis_name="s")


@functools.partial(
    pl.kernel,
    out_type=jax.ShapeDtypeStruct((HIST, BATCH, EMBED), jnp.float32),
    mesh=_mesh,
    scratch_types=[
        pltpu.VMEM((HIST, COLS_PER_W), jnp.int32),
        pltpu.VMEM((NBUF, COLS_PER_W, EMBED), jnp.float32),
        pltpu.SemaphoreType.DMA,
        pltpu.SemaphoreType.DMA,
    ],
)
def _emb_gather(idx_hbm, table_hbm, out_hbm, idx_v, rows_v, gsem, ssem):
    wid = lax.axis_index("s") * 2 + lax.axis_index("c")
    base = wid * COLS_PER_W
    pltpu.sync_copy(idx_hbm.at[:, pl.ds(base, COLS_PER_W)], idx_v)

    def g_copy(h):
        return pltpu.make_async_copy(
            table_hbm.at[idx_v.at[h]], rows_v.at[h % NBUF], gsem)

    def s_copy(h):
        return pltpu.make_async_copy(
            rows_v.at[h % NBUF], out_hbm.at[h, pl.ds(base, COLS_PER_W)], ssem)

    for h in range(NBUF - 1):
        g_copy(h).start()

    @pl.loop(0, HIST)
    def _body(h):
        @pl.when(h > 0)
        def _():
            s_copy(h - 1).wait()

        @pl.when(h + NBUF - 1 < HIST)
        def _():
            g_copy(h + NBUF - 1).start()

        g_copy(h).wait()
        s_copy(h).start()

    s_copy(HIST - 1).wait()


def kernel(x, table):
    xt = x.astype(jnp.int32).T
    out_t = _emb_gather(xt, table)
    return jnp.transpose(out_t, (1, 0, 2))


# speedup vs baseline: 1.0047x; 1.0047x over previous
import functools

import jax
import jax.numpy as jnp
from jax import lax
from jax.experimental import pallas as pl
from jax.experimental.pallas import tpu as pltpu
from jax.experimental.pallas import tpu_sc as plsc

BATCH = 4096
HIST = 50
EMBED = 128
NUM_WORKERS = 32
COLS_PER_W = BATCH // NUM_WORKERS
NBUF = 4

_mesh = plsc.VectorSubcoreMesh(core_axis_name="c", subcore_axis_name="s")


@functools.partial(
    pl.kernel,
    out_type=jax.ShapeDtypeStruct((HIST, BATCH, EMBED), jnp.float32),
    mesh=_mesh,
    scratch_types=[
        pltpu.VMEM((HIST, COLS_PER_W), jnp.int32),
        pltpu.VMEM((NBUF, COLS_PER_W, EMBED), jnp.float32),
        pltpu.SemaphoreType.DMA,
        pltpu.SemaphoreType.DMA,
    ],
)
def _emb_gather(idx_hbm, table_hbm, out_hbm, idx_v, rows_v, gsem, ssem):
    wid = lax.axis_index("s") * 2 + lax.axis_index("c")
    base = wid * COLS_PER_W
    pltpu.sync_copy(idx_hbm.at[:, pl.ds(base, COLS_PER_W)], idx_v)

    def g_copy(h):
        return pltpu.make_async_copy(
            table_hbm.at[idx_v.at[h]], rows_v.at[h % NBUF], gsem)

    def s_copy(h):
        return pltpu.make_async_copy(
            rows_v.at[h % NBUF], out_hbm.at[h, pl.ds(base, COLS_PER_W)], ssem)

    for h in range(NBUF - 1):
        g_copy(h).start()

    @pl.loop(0, HIST)
    def _body(h):
        @pl.when(h > 0)
        def _():
            s_copy(h - 1).wait()

        @pl.when(h + NBUF - 1 < HIST)
        def _():
            g_copy(h + NBUF - 1).start()

        g_copy(h).wait()
        s_copy(h).start()

    s_copy(HIST - 1).wait()


def kernel(x, table):
    xt = x.astype(jnp.int32).T
    out_t = _emb_gather(xt, table)
    return jnp.transpose(out_t, (1, 0, 2))


# speedup vs baseline: 1.0098x; 1.0050x over previous
import functools

import jax
import jax.numpy as jnp
from jax import lax
from jax.experimental import pallas as pl
from jax.experimental.pallas import tpu as pltpu
from jax.experimental.pallas import tpu_sc as plsc

BATCH = 4096
HIST = 50
EMBED = 128
NUM_WORKERS = 32
COLS_PER_W = BATCH // NUM_WORKERS
NBUF = 4

_mesh = plsc.VectorSubcoreMesh(core_axis_name="c", subcore_axis_name="s")


@functools.partial(
    pl.kernel,
    out_type=jax.ShapeDtypeStruct((HIST, BATCH, EMBED), jnp.float32),
    mesh=_mesh,
    scratch_types=[
        pltpu.VMEM((HIST, COLS_PER_W), jnp.int32),
        pltpu.VMEM((NBUF, COLS_PER_W, EMBED), jnp.float32),
        pltpu.SemaphoreType.DMA,
        pltpu.SemaphoreType.DMA,
    ],
)
def _emb_gather(idx_hbm, table_hbm, out_hbm, idx_v, rows_v, gsem, ssem):
    wid = lax.axis_index("s") * 2 + lax.axis_index("c")
    base = wid * COLS_PER_W
    pltpu.sync_copy(idx_hbm.at[pl.ds(0, 8), pl.ds(base, COLS_PER_W)],
                    idx_v.at[pl.ds(0, 8)])

    def g_copy(h):
        return pltpu.make_async_copy(
            table_hbm.at[idx_v.at[h]], rows_v.at[h % NBUF], gsem)

    def s_copy(h):
        return pltpu.make_async_copy(
            rows_v.at[h % NBUF], out_hbm.at[h, pl.ds(base, COLS_PER_W)], ssem)

    for h in range(NBUF - 1):
        g_copy(h).start()
    pltpu.sync_copy(idx_hbm.at[pl.ds(8, HIST - 8), pl.ds(base, COLS_PER_W)],
                    idx_v.at[pl.ds(8, HIST - 8)])

    @pl.loop(0, HIST)
    def _body(h):
        @pl.when(h > 0)
        def _():
            s_copy(h - 1).wait()

        @pl.when(h + NBUF - 1 < HIST)
        def _():
            g_copy(h + NBUF - 1).start()

        g_copy(h).wait()
        s_copy(h).start()

    s_copy(HIST - 1).wait()


def kernel(x, table):
    xt = x.astype(jnp.int32).T
    out_t = _emb_gather(xt, table)
    return jnp.transpose(out_t, (1, 0, 2))
